# baseline (speedup 1.0000x reference)
import functools

import jax
import jax.numpy as jnp
from jax.experimental import pallas as pl
from jax.experimental.pallas import tpu as pltpu

LAMBDA_FAKE_W = 1.0
B = 256
D = 150528
BLK = 6272
NSTEPS = D // BLK


def _loss_kernel(y_ref, r_ref, x_ref, o_ref, acc_ref):
    step = pl.program_id(0)

    @pl.when(step == 0)
    def _init():
        acc_ref[0] = 0.0
        acc_ref[1] = 0.0

    d = jnp.abs(r_ref[...] - x_ref[...])
    s = jnp.sum(d, axis=1, keepdims=True)
    yv = y_ref[...].astype(jnp.float32)
    s_all = jnp.sum(s)
    s_real = jnp.sum(s * yv)
    acc_ref[0] += s_real
    acc_ref[1] += s_all

    @pl.when(step == NSTEPS - 1)
    def _finalize():
        n_real = jnp.sum(y_ref[...].astype(jnp.float32))
        n_fake = B - n_real
        sum_real = acc_ref[0]
        sum_fake = acc_ref[1] - sum_real
        loss_real = jnp.where(n_real > 0, sum_real / (n_real * D), 0.0)
        loss_fake = jnp.where(n_fake > 0, sum_fake / (n_fake * D), 0.0)
        o_ref[...] = (loss_real - LAMBDA_FAKE_W * loss_fake).reshape(1, 1)


def kernel(recons, x, y):
    r2 = recons.reshape(B, D)
    x2 = x.reshape(B, D)
    y2 = y.astype(jnp.float32).reshape(B, 1)

    out = pl.pallas_call(
        _loss_kernel,
        grid=(NSTEPS,),
        in_specs=[
            pl.BlockSpec((B, 1), lambda i: (0, 0)),
            pl.BlockSpec((B, BLK), lambda i: (0, i)),
            pl.BlockSpec((B, BLK), lambda i: (0, i)),
        ],
        out_specs=pl.BlockSpec((1, 1), lambda i: (0, 0)),
        out_shape=jax.ShapeDtypeStruct((1, 1), jnp.float32),
        scratch_shapes=[pltpu.SMEM((2,), jnp.float32)],
        compiler_params=pltpu.CompilerParams(
            dimension_semantics=("arbitrary",),
        ),
    )(y2, r2, x2)
    return out.reshape(())


# speedup vs baseline: 1.0111x; 1.0111x over previous
import jax
import jax.numpy as jnp
from jax.experimental import pallas as pl
from jax.experimental.pallas import tpu as pltpu

LAMBDA_FAKE_W = 1.0
B = 256
D = 150528
RB = 8
NSTEPS = B // RB


def _loss_kernel(y_ref, r_ref, x_ref, o_ref, acc_ref):
    step = pl.program_id(0)

    @pl.when(step == 0)
    def _init():
        acc_ref[0] = 0.0
        acc_ref[1] = 0.0
        acc_ref[2] = 0.0

    d = jnp.abs(r_ref[...] - x_ref[...])
    s = jnp.sum(d, axis=1, keepdims=True)
    yv = y_ref[...].astype(jnp.float32)
    acc_ref[0] += jnp.sum(s * yv)
    acc_ref[1] += jnp.sum(s)
    acc_ref[2] += jnp.sum(yv)

    @pl.when(step == NSTEPS - 1)
    def _finalize():
        n_real = acc_ref[2]
        n_fake = B - n_real
        sum_real = acc_ref[0]
        sum_fake = acc_ref[1] - sum_real
        loss_real = jnp.where(n_real > 0, sum_real / (n_real * D), 0.0)
        loss_fake = jnp.where(n_fake > 0, sum_fake / (n_fake * D), 0.0)
        o_ref[...] = (loss_real - LAMBDA_FAKE_W * loss_fake).reshape(1, 1)


def kernel(recons, x, y):
    r2 = recons.reshape(B, D)
    x2 = x.reshape(B, D)
    y2 = y.astype(jnp.float32).reshape(B, 1)

    out = pl.pallas_call(
        _loss_kernel,
        grid=(NSTEPS,),
        in_specs=[
            pl.BlockSpec((RB, 1), lambda i: (i, 0)),
            pl.BlockSpec((RB, D), lambda i: (i, 0)),
            pl.BlockSpec((RB, D), lambda i: (i, 0)),
        ],
        out_specs=pl.BlockSpec((1, 1), lambda i: (0, 0)),
        out_shape=jax.ShapeDtypeStruct((1, 1), jnp.float32),
        scratch_shapes=[pltpu.SMEM((3,), jnp.float32)],
        compiler_params=pltpu.CompilerParams(
            dimension_semantics=("arbitrary",),
        ),
    )(y2, r2, x2)
    return out.reshape(())
